# Initial kernel scaffold; baseline (speedup 1.0000x reference)
import jax
import jax.numpy as jnp
from jax.experimental import pallas as pl


def kernel(x, batch, ln_weight, ln_bias):
    raise NotImplementedError("write your pallas kernel here")


# speedup vs baseline: 6.5212x; 6.5212x over previous
import jax
import jax.numpy as jnp
from jax import lax
from jax.experimental import pallas as pl
from jax.experimental.pallas import tpu as pltpu
from jax.experimental.pallas import tpu_sc as plsc

N_NODES = 100000
D = 128
NUM_SEGMENTS = 1024
EPS = 1e-5

NC = 2
NS = 16
NW = NC * NS
R = 800
NCHUNK = N_NODES // R
SCW = 80
NSC = R // SCW
CNTW = 16


def _sc_partial_sums(x, batch3, ones_hbm, z_sums, z_cnt):
    mesh = plsc.VectorSubcoreMesh(core_axis_name="c", subcore_axis_name="s")

    @pl.kernel(
        out_type=[
            jax.ShapeDtypeStruct((NC, NUM_SEGMENTS, D), jnp.float32),
            jax.ShapeDtypeStruct((NC, NUM_SEGMENTS, CNTW), jnp.float32),
        ],
        mesh=mesh,
        scratch_types=[
            pltpu.VMEM((R, D), jnp.float32),
            pltpu.VMEM((NSC, SCW), jnp.int32),
            pltpu.VMEM((SCW, CNTW), jnp.float32),
            pltpu.VMEM_SHARED((NUM_SEGMENTS, D), jnp.float32),
            pltpu.VMEM_SHARED((NUM_SEGMENTS, CNTW), jnp.float32),
        ],
    )
    def k(x_hbm, b_hbm, ones_h, zs_h, zc_h, sums_out, cnts_out,
          xbuf, idxbuf, onesbuf, sums_sh, cnts_sh):
        cid = lax.axis_index("c")
        sid = lax.axis_index("s")
        wid = sid * NC + cid

        @pl.when(sid == 0)
        def _():
            pltpu.sync_copy(zs_h, sums_sh)
            pltpu.sync_copy(zc_h, cnts_sh)

        pltpu.sync_copy(ones_h, onesbuf)
        plsc.subcore_barrier()

        lo = (wid * NCHUNK) // NW
        hi = ((wid + 1) * NCHUNK) // NW

        def body(i, carry):
            pltpu.sync_copy(x_hbm.at[pl.ds(i * R, R)], xbuf)
            pltpu.sync_copy(b_hbm.at[i], idxbuf)
            for j in range(NSC):
                pltpu.sync_copy(xbuf.at[pl.ds(j * SCW, SCW)],
                                sums_sh.at[idxbuf.at[j]], add=True)
                pltpu.sync_copy(onesbuf, cnts_sh.at[idxbuf.at[j]], add=True)
            return carry

        lax.fori_loop(lo, hi, body, 0)
        plsc.subcore_barrier()

        @pl.when(sid == 0)
        def _():
            pltpu.sync_copy(sums_sh, sums_out.at[cid])
            pltpu.sync_copy(cnts_sh, cnts_out.at[cid])

    return k(x, batch3, ones_hbm, z_sums, z_cnt)


def _finish(sums_ref, cnts_ref, w_ref, b_ref, o_ref):
    s = sums_ref[0] + sums_ref[1]
    c = cnts_ref[0, :, 0:1] + cnts_ref[1, :, 0:1]
    h = s / jnp.maximum(c, 1.0)
    mu = jnp.mean(h, axis=1, keepdims=True)
    var = jnp.mean((h - mu) ** 2, axis=1, keepdims=True)
    o_ref[...] = (h - mu) * lax.rsqrt(var + EPS) * w_ref[0] + b_ref[0]


def kernel(x, batch, ln_weight, ln_bias):
    batch3 = batch.astype(jnp.int32).reshape(NCHUNK, NSC, SCW)
    ones_hbm = jnp.ones((SCW, CNTW), dtype=jnp.float32)
    z_sums = jnp.zeros((NUM_SEGMENTS, D), dtype=jnp.float32)
    z_cnt = jnp.zeros((NUM_SEGMENTS, CNTW), dtype=jnp.float32)

    sums_p, cnts_p = _sc_partial_sums(x, batch3, ones_hbm, z_sums, z_cnt)

    return pl.pallas_call(
        _finish,
        out_shape=jax.ShapeDtypeStruct((NUM_SEGMENTS, D), jnp.float32),
    )(sums_p, cnts_p, ln_weight.reshape(1, D), ln_bias.reshape(1, D))


# speedup vs baseline: 7.3412x; 1.1257x over previous
import jax
import jax.numpy as jnp
from jax import lax
from jax.experimental import pallas as pl
from jax.experimental.pallas import tpu as pltpu
from jax.experimental.pallas import tpu_sc as plsc

N_NODES = 100000
D = 128
NUM_SEGMENTS = 1024
EPS = 1e-5

NC = 2
NS = 16
NW = NC * NS
R = 400
NCHUNK = N_NODES // R
SCW = 80
NSC = R // SCW
CNTW = 16


def _sc_partial_sums(x, batch3, ones_hbm, z_sums, z_cnt):
    mesh = plsc.VectorSubcoreMesh(core_axis_name="c", subcore_axis_name="s")

    @pl.kernel(
        out_type=[
            jax.ShapeDtypeStruct((NC, NUM_SEGMENTS, D), jnp.float32),
            jax.ShapeDtypeStruct((NC, NUM_SEGMENTS, CNTW), jnp.float32),
        ],
        mesh=mesh,
        scratch_types=[
            pltpu.VMEM((2, R, D), jnp.float32),
            pltpu.VMEM((2, NSC, SCW), jnp.int32),
            pltpu.VMEM((SCW, CNTW), jnp.float32),
            pltpu.VMEM_SHARED((NUM_SEGMENTS, D), jnp.float32),
            pltpu.VMEM_SHARED((NUM_SEGMENTS, CNTW), jnp.float32),
            pltpu.SemaphoreType.DMA,
            pltpu.SemaphoreType.DMA,
            pltpu.SemaphoreType.DMA,
        ],
    )
    def k(x_hbm, b_hbm, ones_h, zs_h, zc_h, sums_out, cnts_out,
          xbuf, idxbuf, onesbuf, sums_sh, cnts_sh, sem0, sem1, sem_sc):
        cid = lax.axis_index("c")
        sid = lax.axis_index("s")
        wid = sid * NC + cid
        sems = (sem0, sem1)

        @pl.when(sid == 0)
        def _():
            pltpu.sync_copy(zs_h, sums_sh)
            pltpu.sync_copy(zc_h, cnts_sh)

        pltpu.sync_copy(ones_h, onesbuf)
        plsc.subcore_barrier()

        lo = (wid * NCHUNK) // NW
        hi = ((wid + 1) * NCHUNK) // NW

        def start_in(chunk, b):
            pltpu.async_copy(x_hbm.at[pl.ds(chunk * R, R)], xbuf.at[b], sems[b])
            pltpu.async_copy(b_hbm.at[chunk], idxbuf.at[b], sems[b])

        def wait_in(chunk, b):
            pltpu.make_async_copy(x_hbm.at[pl.ds(chunk * R, R)], xbuf.at[b],
                                  sems[b]).wait()
            pltpu.make_async_copy(b_hbm.at[chunk], idxbuf.at[b],
                                  sems[b]).wait()

        start_in(lo, 0)
        start_in(lo + 1, 1)

        n_outer = (hi - lo + 1) // 2

        def body(kk, carry):
            i = lo + 2 * kk
            for b in range(2):
                chunk = i + b

                @pl.when(chunk < hi)
                def _():
                    wait_in(chunk, b)
                    xv = xbuf.at[b]
                    iv = idxbuf.at[b]
                    descs = []
                    for j in range(NSC):
                        descs.append(pltpu.async_copy(
                            xv.at[pl.ds(j * SCW, SCW)],
                            sums_sh.at[iv.at[j]], sem_sc, add=True))
                        descs.append(pltpu.async_copy(
                            onesbuf, cnts_sh.at[iv.at[j]], sem_sc, add=True))
                    for d in descs:
                        d.wait()

                    @pl.when(chunk + 2 < hi)
                    def _():
                        start_in(chunk + 2, b)
            return carry

        lax.fori_loop(0, n_outer, body, 0)
        plsc.subcore_barrier()

        @pl.when(sid == 0)
        def _():
            pltpu.sync_copy(sums_sh, sums_out.at[cid])
            pltpu.sync_copy(cnts_sh, cnts_out.at[cid])

    return k(x, batch3, ones_hbm, z_sums, z_cnt)


def _finish(sums_ref, cnts_ref, w_ref, b_ref, o_ref):
    s = sums_ref[0] + sums_ref[1]
    c = cnts_ref[0, :, 0:1] + cnts_ref[1, :, 0:1]
    h = s / jnp.maximum(c, 1.0)
    mu = jnp.mean(h, axis=1, keepdims=True)
    var = jnp.mean((h - mu) ** 2, axis=1, keepdims=True)
    o_ref[...] = (h - mu) * lax.rsqrt(var + EPS) * w_ref[0] + b_ref[0]


def kernel(x, batch, ln_weight, ln_bias):
    batch3 = batch.astype(jnp.int32).reshape(NCHUNK, NSC, SCW)
    ones_hbm = jnp.ones((SCW, CNTW), dtype=jnp.float32)
    z_sums = jnp.zeros((NUM_SEGMENTS, D), dtype=jnp.float32)
    z_cnt = jnp.zeros((NUM_SEGMENTS, CNTW), dtype=jnp.float32)

    sums_p, cnts_p = _sc_partial_sums(x, batch3, ones_hbm, z_sums, z_cnt)

    return pl.pallas_call(
        _finish,
        out_shape=jax.ShapeDtypeStruct((NUM_SEGMENTS, D), jnp.float32),
    )(sums_p, cnts_p, ln_weight.reshape(1, D), ln_bias.reshape(1, D))
